# baseline (speedup 1.0000x reference)
import functools

import jax
import jax.numpy as jnp
from jax import lax
from jax.experimental import pallas as pl
from jax.experimental.pallas import tpu as pltpu
from jax.experimental.pallas import tpu_sc as plsc

RESO = 128
OUT = 3
G_RAD = 1.3
T_NEAR = 0.2
T_FAR = 2.0
STEPS = 64
VOXEL_LEN = G_RAD * 2 / RESO
N_RAYS = 4096
EPS = 1e-10

P = STEPS * N_RAYS
NW = 32
PPW = P // NW
CH = 512
NCHUNK = PPW // CH
NBLK = CH // 128


def _tc_prep(rays_t, ts2):
    JB = 8

    def body(rays_ref, ts_ref, base_ref, slot_ref, w_ref):
        t = ts_ref[...]
        pts = []
        for d in range(3):
            ro = rays_ref[d:d + 1, :]
            rd = rays_ref[d + 3:d + 4, :]
            pts.append(ro + t * rd)
        ilo, ihi, tx = [], [], []
        for d in range(3):
            p = pts[d]
            nlo = jnp.clip(-0.5 * VOXEL_LEN + p, -G_RAD, G_RAD)
            nhi = jnp.clip(0.5 * VOXEL_LEN + p, -G_RAD, G_RAD)
            clo = jnp.clip((jnp.floor(nlo / VOXEL_LEN + EPS) + 0.5) * VOXEL_LEN,
                           -(G_RAD - VOXEL_LEN / 2), G_RAD - VOXEL_LEN / 2)
            chi = jnp.clip((jnp.floor(nhi / VOXEL_LEN + EPS) + 0.5) * VOXEL_LEN,
                           -(G_RAD - VOXEL_LEN / 2), G_RAD - VOXEL_LEN / 2)
            ilo.append(jnp.floor(clo / VOXEL_LEN + EPS).astype(jnp.int32) + RESO // 2)
            ihi.append(jnp.floor(chi / VOXEL_LEN + EPS).astype(jnp.int32) + RESO // 2)
            x = (p - clo) / VOXEL_LEN
            tx.append((1 - x, x))
        zbase = jnp.minimum(ilo[2], RESO - 2)
        for cu in range(4):
            bx, by = cu & 1, (cu >> 1) & 1
            ix = ihi[0] if bx else ilo[0]
            iy = ihi[1] if by else ilo[1]
            base_ref[cu] = (ix * RESO + iy) * RESO + zbase
        for u in range(8):
            bx, by, bz = u & 1, (u >> 1) & 1, (u >> 2) & 1
            iz = ihi[2] if bz else ilo[2]
            slot_ref[u] = jnp.where(iz == zbase, 0, 4).astype(jnp.int32)
            w_ref[u] = tx[0][bx] * tx[1][by] * tx[2][bz]

    return pl.pallas_call(
        body,
        grid=(STEPS // JB,),
        in_specs=[
            pl.BlockSpec((6, N_RAYS), lambda j: (0, 0)),
            pl.BlockSpec((JB, 1), lambda j: (j, 0)),
        ],
        out_specs=[
            pl.BlockSpec((4, JB, N_RAYS), lambda j: (0, j, 0)),
            pl.BlockSpec((8, JB, N_RAYS), lambda j: (0, j, 0)),
            pl.BlockSpec((8, JB, N_RAYS), lambda j: (0, j, 0)),
        ],
        out_shape=[
            jax.ShapeDtypeStruct((4, STEPS, N_RAYS), jnp.int32),
            jax.ShapeDtypeStruct((8, STEPS, N_RAYS), jnp.int32),
            jax.ShapeDtypeStruct((8, STEPS, N_RAYS), jnp.float32),
        ],
    )(rays_t, ts2)


def _sc_gather(table2, base4, slot8, w8):
    mesh = plsc.VectorSubcoreMesh(core_axis_name="c", subcore_axis_name="s")

    @functools.partial(
        pl.kernel,
        mesh=mesh,
        compiler_params=pltpu.CompilerParams(
            needs_layout_passes=False, use_tc_tiling_on_sc=False),
        out_type=jax.ShapeDtypeStruct((4, P), jnp.float32),
        scratch_types=[
            pltpu.VMEM((4, NBLK, 128), jnp.int32),
            pltpu.VMEM((8, NBLK, 128), jnp.int32),
            pltpu.VMEM((8, NBLK, 128), jnp.float32),
            pltpu.VMEM((4, NBLK, 128, 8), jnp.float32),
            pltpu.VMEM((4, CH), jnp.float32),
            pltpu.SemaphoreType.DMA,
        ],
    )
    def k(tab_hbm, base_hbm, slot_hbm, w_hbm, out_hbm,
          base_v, slot_v, w_v, rows_v, out_v, gsem):
        wid = lax.axis_index("s") * 2 + lax.axis_index("c")
        base_blk = wid * (PPW // 128)
        iota = lax.iota(jnp.int32, 16)
        lvecs = [iota + m * 16 for m in range(8)]

        def chunk(ci, carry):
            blk = base_blk + ci * NBLK
            pltpu.sync_copy(base_hbm.at[:, pl.ds(blk, NBLK)], base_v)
            pltpu.sync_copy(slot_hbm.at[:, pl.ds(blk, NBLK)], slot_v)
            pltpu.sync_copy(w_hbm.at[:, pl.ds(blk, NBLK)], w_v)
            handles = []
            for cu in range(4):
                for kb in range(NBLK):
                    handles.append(pltpu.async_copy(
                        tab_hbm.at[base_v.at[cu, kb]], rows_v.at[cu, kb], gsem))
            for h in handles:
                h.wait()
            for g in range(CH // 16):
                kb = g // 8
                lvec = lvecs[g % 8]
                kbv = jnp.full((16,), kb, jnp.int32)
                accs = [jnp.zeros((16,), jnp.float32) for _ in range(4)]
                for u in range(8):
                    uv = jnp.full((16,), u, jnp.int32)
                    cuv = jnp.full((16,), u & 3, jnp.int32)
                    slotv = plsc.load_gather(slot_v, [uv, kbv, lvec])
                    wv = plsc.load_gather(w_v, [uv, kbv, lvec])
                    for c in range(4):
                        val = plsc.load_gather(
                            rows_v, [cuv, kbv, lvec, slotv + c])
                        prod = wv * val
                        accs[c] = accs[c] + prod
                for c in range(4):
                    out_v[c, pl.ds(g * 16, 16)] = accs[c]
            pt0 = wid * PPW + ci * CH
            pltpu.sync_copy(out_v, out_hbm.at[:, pl.ds(pt0, CH)])
            return carry

        lax.fori_loop(0, NCHUNK, chunk, 0)

    return k(table2, base4, slot8, w8)


def _tc_finish(acc4, rays_t, dt):
    JB = 8

    def body(acc_ref, rays_ref, dt_ref, o_ref):
        dens = acc_ref[0]
        rx = rays_ref[3:4, :]
        ry = rays_ref[4:5, :]
        rz = rays_ref[5:6, :]
        norm = jnp.sqrt(rx * rx + ry * ry + rz * rz)
        dt_s = dt_ref[0, 0]
        sigma_a = jax.nn.softplus(dens - 1)
        a = 1 - jnp.exp(-sigma_a * (dt_s * norm))
        b = 1 - jnp.exp(-sigma_a * (1e10 * norm))
        q = 1 - a + 1e-10
        q2 = q * q
        q4 = q2 * q2
        q8 = q4 * q4
        q16 = q8 * q8
        q32 = q16 * q16
        q63 = q32 * q16 * q8 * q4 * q2 * q
        s63 = jnp.where(jnp.abs(1 - q) > 1e-9, (1 - q63) / (1 - q), 63.0)
        o_ref[...] = a * s63 + b * q63

    return pl.pallas_call(
        body,
        grid=(STEPS // JB,),
        in_specs=[
            pl.BlockSpec((4, JB, N_RAYS), lambda j: (0, j, 0)),
            pl.BlockSpec((6, N_RAYS), lambda j: (0, 0)),
            pl.BlockSpec((1, 1), lambda j: (0, 0), memory_space=pltpu.SMEM),
        ],
        out_specs=pl.BlockSpec((JB, N_RAYS), lambda j: (j, 0)),
        out_shape=jax.ShapeDtypeStruct((STEPS, N_RAYS), jnp.float32),
    )(acc4, rays_t, dt.reshape(1, 1))


def _sc_finalize(acc, wtot):
    mesh = plsc.VectorSubcoreMesh(core_axis_name="c", subcore_axis_name="s")

    @functools.partial(
        pl.kernel,
        mesh=mesh,
        compiler_params=pltpu.CompilerParams(
            needs_layout_passes=False, use_tc_tiling_on_sc=False),
        out_type=jax.ShapeDtypeStruct((1, STEPS, N_RAYS, 3), jnp.float32),
        scratch_types=[
            pltpu.VMEM((CH,), jnp.float32),
            pltpu.VMEM((3, CH), jnp.float32),
            pltpu.VMEM((CH, 3), jnp.float32),
        ],
    )
    def k(acc_hbm, w_hbm, out_hbm, w_v, rgb_v, o_v):
        wid = lax.axis_index("s") * 2 + lax.axis_index("c")
        iota = lax.iota(jnp.int32, 16)

        def chunk(ci, carry):
            jj = 2 * wid + ci // 8
            n0 = (ci % 8) * CH
            pt0 = wid * PPW + ci * CH
            pltpu.sync_copy(w_hbm.at[jj, pl.ds(n0, CH)], w_v)
            pltpu.sync_copy(acc_hbm.at[pl.ds(1, 3), pl.ds(pt0, CH)], rgb_v)
            for g in range(CH // 16):
                lvec = iota + g * 16
                wv = w_v[pl.ds(g * 16, 16)]
                for c in range(3):
                    rv = rgb_v[c, pl.ds(g * 16, 16)]
                    plsc.store_scatter(
                        o_v, [lvec, jnp.full((16,), c, jnp.int32)], wv * rv)
            pltpu.sync_copy(o_v, out_hbm.at[0, jj, pl.ds(n0, CH)])
            return carry

        lax.fori_loop(0, NCHUNK, chunk, 0)

    return k(acc, wtot)


def kernel(rays, densities, rgb):
    ts = jnp.linspace(T_NEAR, T_FAR, STEPS, dtype=rays.dtype)
    dt = jnp.clip(ts[1] - ts[0], 1e-5, None)
    rays_t = rays.T
    act_rgb = jax.nn.sigmoid(rgb) * (1 + 2e-3) - 1e-3
    d_pl = densities.reshape(-1)
    ch_pl = [act_rgb[..., c].reshape(-1) for c in range(OUT)]
    planes = [d_pl] + ch_pl
    table2 = jnp.stack([p[:-1] for p in planes] + [p[1:] for p in planes],
                       axis=-1)

    base4, slot8, w8 = _tc_prep(rays_t, ts[:, None])
    acc = _sc_gather(table2,
                     base4.reshape(4, P // 128, 128),
                     slot8.reshape(8, P // 128, 128),
                     w8.reshape(8, P // 128, 128))
    wtot = _tc_finish(acc.reshape(4, STEPS, N_RAYS), rays_t, dt)
    return _sc_finalize(acc, wtot)


# speedup vs baseline: 2.2140x; 2.2140x over previous
import functools

import jax
import jax.numpy as jnp
from jax import lax
from jax.experimental import pallas as pl
from jax.experimental.pallas import tpu as pltpu
from jax.experimental.pallas import tpu_sc as plsc

RESO = 128
OUT = 3
G_RAD = 1.3
T_NEAR = 0.2
T_FAR = 2.0
STEPS = 64
VOXEL_LEN = G_RAD * 2 / RESO
N_RAYS = 4096
EPS = 1e-10

P = STEPS * N_RAYS
NW = 32
PPW = P // NW
CH = 512
NCHUNK = PPW // CH
NBLK = CH // 128


def _tc_prep(rays_t, ts2):
    JB = 8

    def body(rays_ref, ts_ref, base_ref, slot_ref, w_ref):
        t = ts_ref[...]
        pts = []
        for d in range(3):
            ro = rays_ref[d:d + 1, :]
            rd = rays_ref[d + 3:d + 4, :]
            pts.append(ro + t * rd)
        ilo, ihi, tx = [], [], []
        for d in range(3):
            p = pts[d]
            nlo = jnp.clip(-0.5 * VOXEL_LEN + p, -G_RAD, G_RAD)
            nhi = jnp.clip(0.5 * VOXEL_LEN + p, -G_RAD, G_RAD)
            clo = jnp.clip((jnp.floor(nlo / VOXEL_LEN + EPS) + 0.5) * VOXEL_LEN,
                           -(G_RAD - VOXEL_LEN / 2), G_RAD - VOXEL_LEN / 2)
            chi = jnp.clip((jnp.floor(nhi / VOXEL_LEN + EPS) + 0.5) * VOXEL_LEN,
                           -(G_RAD - VOXEL_LEN / 2), G_RAD - VOXEL_LEN / 2)
            ilo.append(jnp.floor(clo / VOXEL_LEN + EPS).astype(jnp.int32) + RESO // 2)
            ihi.append(jnp.floor(chi / VOXEL_LEN + EPS).astype(jnp.int32) + RESO // 2)
            x = (p - clo) / VOXEL_LEN
            tx.append((1 - x, x))
        zbase = jnp.minimum(ilo[2], RESO - 2)
        for cu in range(4):
            bx, by = cu & 1, (cu >> 1) & 1
            ix = ihi[0] if bx else ilo[0]
            iy = ihi[1] if by else ilo[1]
            base_ref[cu] = (ix * RESO + iy) * RESO + zbase
        for u in range(8):
            bx, by, bz = u & 1, (u >> 1) & 1, (u >> 2) & 1
            iz = ihi[2] if bz else ilo[2]
            slot_ref[u] = jnp.where(iz == zbase, 0, 4).astype(jnp.int32)
            w_ref[u] = tx[0][bx] * tx[1][by] * tx[2][bz]

    return pl.pallas_call(
        body,
        grid=(STEPS // JB,),
        in_specs=[
            pl.BlockSpec((6, N_RAYS), lambda j: (0, 0)),
            pl.BlockSpec((JB, 1), lambda j: (j, 0)),
        ],
        out_specs=[
            pl.BlockSpec((4, JB, N_RAYS), lambda j: (0, j, 0)),
            pl.BlockSpec((8, JB, N_RAYS), lambda j: (0, j, 0)),
            pl.BlockSpec((8, JB, N_RAYS), lambda j: (0, j, 0)),
        ],
        out_shape=[
            jax.ShapeDtypeStruct((4, STEPS, N_RAYS), jnp.int32),
            jax.ShapeDtypeStruct((8, STEPS, N_RAYS), jnp.int32),
            jax.ShapeDtypeStruct((8, STEPS, N_RAYS), jnp.float32),
        ],
    )(rays_t, ts2)


def _sc_build(d_flat, act_flat):
    PL = RESO * RESO
    C2 = 2048
    NC = (RESO // NW) * (PL // C2)
    mesh = plsc.VectorSubcoreMesh(core_axis_name="c", subcore_axis_name="s")

    @functools.partial(
        pl.kernel,
        mesh=mesh,
        compiler_params=pltpu.CompilerParams(
            needs_layout_passes=False, use_tc_tiling_on_sc=False),
        out_type=jax.ShapeDtypeStruct((RESO ** 3, 8), jnp.float32),
        scratch_types=[
            pltpu.VMEM((C2 + 8,), jnp.float32),
            pltpu.VMEM((3, C2 + 8), jnp.float32),
            pltpu.VMEM((C2, 8), jnp.float32),
        ],
    )
    def k(d_hbm, a_hbm, t2_hbm, dbuf, abuf, t2buf):
        wid = lax.axis_index("s") * 2 + lax.axis_index("c")
        iota = lax.iota(jnp.int32, 16)

        def chunk(i, carry):
            plane = wid * (RESO // NW) + i // (PL // C2)
            koff = (i % (PL // C2)) * C2
            row0 = plane * PL + koff
            pltpu.sync_copy(d_hbm.at[pl.ds(row0, C2)], dbuf.at[pl.ds(0, C2)])
            dlo = jnp.minimum(row0 + C2, RESO ** 3 - 8)
            pltpu.sync_copy(d_hbm.at[pl.ds(dlo, 8)], dbuf.at[pl.ds(C2, 8)])
            for c in range(3):
                aoff = (plane * 3 + c) * PL + koff
                pltpu.sync_copy(a_hbm.at[pl.ds(aoff, C2)],
                                abuf.at[c, pl.ds(0, C2)])
                alo = jnp.minimum(aoff + C2, RESO ** 3 * 3 - 8)
                pltpu.sync_copy(a_hbm.at[pl.ds(alo, 8)],
                                abuf.at[c, pl.ds(C2, 8)])
            for g in range(C2 // 16):
                lvec = iota + g * 16
                vecs = [dbuf[pl.ds(g * 16, 16)]]
                vecs += [abuf[c, pl.ds(g * 16, 16)] for c in range(3)]
                vecs += [dbuf[pl.ds(g * 16 + 1, 16)]]
                vecs += [abuf[c, pl.ds(g * 16 + 1, 16)] for c in range(3)]
                for s in range(8):
                    plsc.store_scatter(
                        t2buf, [lvec, jnp.full((16,), s, jnp.int32)], vecs[s])
            pltpu.sync_copy(t2buf, t2_hbm.at[pl.ds(row0, C2)])
            return carry

        lax.fori_loop(0, NC, chunk, 0)

    return k(d_flat, act_flat)


def _sc_gather(table2, base4, slot8, w8):
    mesh = plsc.VectorSubcoreMesh(core_axis_name="c", subcore_axis_name="s")

    @functools.partial(
        pl.kernel,
        mesh=mesh,
        compiler_params=pltpu.CompilerParams(
            needs_layout_passes=False, use_tc_tiling_on_sc=False),
        out_type=jax.ShapeDtypeStruct((4, P), jnp.float32),
        scratch_types=[
            pltpu.VMEM((4, NBLK, 128), jnp.int32),
            pltpu.VMEM((8, NBLK, 128), jnp.int32),
            pltpu.VMEM((8, NBLK, 128), jnp.float32),
            pltpu.VMEM((4, NBLK, 128, 8), jnp.float32),
            pltpu.VMEM((4, CH), jnp.float32),
            pltpu.SemaphoreType.DMA,
        ],
    )
    def k(tab_hbm, base_hbm, slot_hbm, w_hbm, out_hbm,
          base_v, slot_v, w_v, rows_v, out_v, gsem):
        wid = lax.axis_index("s") * 2 + lax.axis_index("c")
        base_blk = wid * (PPW // 128)
        iota = lax.iota(jnp.int32, 16)
        lvecs = [iota + m * 16 for m in range(8)]

        def chunk(ci, carry):
            blk = base_blk + ci * NBLK
            pltpu.sync_copy(base_hbm.at[:, pl.ds(blk, NBLK)], base_v)
            pltpu.sync_copy(slot_hbm.at[:, pl.ds(blk, NBLK)], slot_v)
            pltpu.sync_copy(w_hbm.at[:, pl.ds(blk, NBLK)], w_v)
            handles = []
            for cu in range(4):
                for kb in range(NBLK):
                    handles.append(pltpu.async_copy(
                        tab_hbm.at[base_v.at[cu, kb]], rows_v.at[cu, kb], gsem))
            for h in handles:
                h.wait()
            for g in range(CH // 16):
                kb = g // 8
                lvec = lvecs[g % 8]
                kbv = jnp.full((16,), kb, jnp.int32)
                accs = [jnp.zeros((16,), jnp.float32) for _ in range(4)]
                for u in range(8):
                    uv = jnp.full((16,), u, jnp.int32)
                    cuv = jnp.full((16,), u & 3, jnp.int32)
                    slotv = plsc.load_gather(slot_v, [uv, kbv, lvec])
                    wv = plsc.load_gather(w_v, [uv, kbv, lvec])
                    for c in range(4):
                        val = plsc.load_gather(
                            rows_v, [cuv, kbv, lvec, slotv + c])
                        prod = wv * val
                        accs[c] = accs[c] + prod
                for c in range(4):
                    out_v[c, pl.ds(g * 16, 16)] = accs[c]
            pt0 = wid * PPW + ci * CH
            pltpu.sync_copy(out_v, out_hbm.at[:, pl.ds(pt0, CH)])
            return carry

        lax.fori_loop(0, NCHUNK, chunk, 0)

    return k(table2, base4, slot8, w8)


def _tc_finish(acc4, rays_t, dt):
    JB = 8

    def body(acc_ref, rays_ref, dt_ref, o_ref):
        dens = acc_ref[0]
        rx = rays_ref[3:4, :]
        ry = rays_ref[4:5, :]
        rz = rays_ref[5:6, :]
        norm = jnp.sqrt(rx * rx + ry * ry + rz * rz)
        dt_s = dt_ref[0, 0]
        sigma_a = jax.nn.softplus(dens - 1)
        a = 1 - jnp.exp(-sigma_a * (dt_s * norm))
        b = 1 - jnp.exp(-sigma_a * (1e10 * norm))
        q = 1 - a + 1e-10
        q2 = q * q
        q4 = q2 * q2
        q8 = q4 * q4
        q16 = q8 * q8
        q32 = q16 * q16
        q63 = q32 * q16 * q8 * q4 * q2 * q
        s63 = jnp.where(jnp.abs(1 - q) > 1e-9, (1 - q63) / (1 - q), 63.0)
        o_ref[...] = a * s63 + b * q63

    return pl.pallas_call(
        body,
        grid=(STEPS // JB,),
        in_specs=[
            pl.BlockSpec((4, JB, N_RAYS), lambda j: (0, j, 0)),
            pl.BlockSpec((6, N_RAYS), lambda j: (0, 0)),
            pl.BlockSpec((1, 1), lambda j: (0, 0), memory_space=pltpu.SMEM),
        ],
        out_specs=pl.BlockSpec((JB, N_RAYS), lambda j: (j, 0)),
        out_shape=jax.ShapeDtypeStruct((STEPS, N_RAYS), jnp.float32),
    )(acc4, rays_t, dt.reshape(1, 1))


def _sc_finalize(acc, wtot):
    mesh = plsc.VectorSubcoreMesh(core_axis_name="c", subcore_axis_name="s")

    @functools.partial(
        pl.kernel,
        mesh=mesh,
        compiler_params=pltpu.CompilerParams(
            needs_layout_passes=False, use_tc_tiling_on_sc=False),
        out_type=jax.ShapeDtypeStruct((1, STEPS, N_RAYS, 3), jnp.float32),
        scratch_types=[
            pltpu.VMEM((CH,), jnp.float32),
            pltpu.VMEM((3, CH), jnp.float32),
            pltpu.VMEM((CH, 3), jnp.float32),
        ],
    )
    def k(acc_hbm, w_hbm, out_hbm, w_v, rgb_v, o_v):
        wid = lax.axis_index("s") * 2 + lax.axis_index("c")
        iota = lax.iota(jnp.int32, 16)

        def chunk(ci, carry):
            jj = 2 * wid + ci // 8
            n0 = (ci % 8) * CH
            pt0 = wid * PPW + ci * CH
            pltpu.sync_copy(w_hbm.at[jj, pl.ds(n0, CH)], w_v)
            pltpu.sync_copy(acc_hbm.at[pl.ds(1, 3), pl.ds(pt0, CH)], rgb_v)
            for g in range(CH // 16):
                lvec = iota + g * 16
                wv = w_v[pl.ds(g * 16, 16)]
                for c in range(3):
                    rv = rgb_v[c, pl.ds(g * 16, 16)]
                    plsc.store_scatter(
                        o_v, [lvec, jnp.full((16,), c, jnp.int32)], wv * rv)
            pltpu.sync_copy(o_v, out_hbm.at[0, jj, pl.ds(n0, CH)])
            return carry

        lax.fori_loop(0, NCHUNK, chunk, 0)

    return k(acc, wtot)


def kernel(rays, densities, rgb):
    ts = jnp.linspace(T_NEAR, T_FAR, STEPS, dtype=rays.dtype)
    dt = jnp.clip(ts[1] - ts[0], 1e-5, None)
    rays_t = rays.T
    act_rgb = jax.nn.sigmoid(rgb) * (1 + 2e-3) - 1e-3
    d_flat = densities.reshape(-1)
    act_flat = jnp.transpose(act_rgb, (0, 3, 1, 2)).reshape(-1)
    table2 = _sc_build(d_flat, act_flat)

    base4, slot8, w8 = _tc_prep(rays_t, ts[:, None])
    acc = _sc_gather(table2,
                     base4.reshape(4, P // 128, 128),
                     slot8.reshape(8, P // 128, 128),
                     w8.reshape(8, P // 128, 128))
    wtot = _tc_finish(acc.reshape(4, STEPS, N_RAYS), rays_t, dt)
    return _sc_finalize(acc, wtot)


# speedup vs baseline: 2.5225x; 1.1393x over previous
import functools

import jax
import jax.numpy as jnp
from jax import lax
from jax.experimental import pallas as pl
from jax.experimental.pallas import tpu as pltpu
from jax.experimental.pallas import tpu_sc as plsc

RESO = 128
OUT = 3
G_RAD = 1.3
T_NEAR = 0.2
T_FAR = 2.0
STEPS = 64
VOXEL_LEN = G_RAD * 2 / RESO
N_RAYS = 4096
EPS = 1e-10

P = STEPS * N_RAYS
NW = 32
PPW = P // NW
CH = 512
NCHUNK = PPW // CH
NBLK = CH // 128


def _tc_prep(rays_t, ts2):
    JB = 8

    def body(rays_ref, ts_ref, base_ref, slot_ref, w_ref):
        t = ts_ref[...]
        pts = []
        for d in range(3):
            ro = rays_ref[d:d + 1, :]
            rd = rays_ref[d + 3:d + 4, :]
            pts.append(ro + t * rd)
        ilo, ihi, tx = [], [], []
        for d in range(3):
            p = pts[d]
            nlo = jnp.clip(-0.5 * VOXEL_LEN + p, -G_RAD, G_RAD)
            nhi = jnp.clip(0.5 * VOXEL_LEN + p, -G_RAD, G_RAD)
            clo = jnp.clip((jnp.floor(nlo / VOXEL_LEN + EPS) + 0.5) * VOXEL_LEN,
                           -(G_RAD - VOXEL_LEN / 2), G_RAD - VOXEL_LEN / 2)
            chi = jnp.clip((jnp.floor(nhi / VOXEL_LEN + EPS) + 0.5) * VOXEL_LEN,
                           -(G_RAD - VOXEL_LEN / 2), G_RAD - VOXEL_LEN / 2)
            ilo.append(jnp.floor(clo / VOXEL_LEN + EPS).astype(jnp.int32) + RESO // 2)
            ihi.append(jnp.floor(chi / VOXEL_LEN + EPS).astype(jnp.int32) + RESO // 2)
            x = (p - clo) / VOXEL_LEN
            tx.append((1 - x, x))
        zbase = jnp.minimum(ilo[2], RESO - 2)
        for cu in range(4):
            bx, by = cu & 1, (cu >> 1) & 1
            ix = ihi[0] if bx else ilo[0]
            iy = ihi[1] if by else ilo[1]
            base_ref[cu] = (ix * RESO + iy) * RESO + zbase
        for u in range(8):
            bx, by, bz = u & 1, (u >> 1) & 1, (u >> 2) & 1
            iz = ihi[2] if bz else ilo[2]
            slot_ref[u] = jnp.where(iz == zbase, 0, 4).astype(jnp.int32)
            w_ref[u] = tx[0][bx] * tx[1][by] * tx[2][bz]

    return pl.pallas_call(
        body,
        grid=(STEPS // JB,),
        in_specs=[
            pl.BlockSpec((6, N_RAYS), lambda j: (0, 0)),
            pl.BlockSpec((JB, 1), lambda j: (j, 0)),
        ],
        out_specs=[
            pl.BlockSpec((4, JB, N_RAYS), lambda j: (0, j, 0)),
            pl.BlockSpec((8, JB, N_RAYS), lambda j: (0, j, 0)),
            pl.BlockSpec((8, JB, N_RAYS), lambda j: (0, j, 0)),
        ],
        out_shape=[
            jax.ShapeDtypeStruct((4, STEPS, N_RAYS), jnp.int32),
            jax.ShapeDtypeStruct((8, STEPS, N_RAYS), jnp.int32),
            jax.ShapeDtypeStruct((8, STEPS, N_RAYS), jnp.float32),
        ],
    )(rays_t, ts2)


def _sc_build(d_flat, act_flat):
    PL = RESO * RESO
    C2 = 2048
    NC = (RESO // NW) * (PL // C2)
    mesh = plsc.VectorSubcoreMesh(core_axis_name="c", subcore_axis_name="s")

    del NC

    @functools.partial(
        pl.kernel,
        mesh=mesh,
        compiler_params=pltpu.CompilerParams(
            needs_layout_passes=False, use_tc_tiling_on_sc=False),
        out_type=jax.ShapeDtypeStruct((RESO ** 3, 8), jnp.float32),
        scratch_types=[
            pltpu.VMEM((PL + 8,), jnp.float32),
            pltpu.VMEM((3, PL + 8), jnp.float32),
            pltpu.VMEM((2, C2, 8), jnp.float32),
            pltpu.SemaphoreType.DMA,
            pltpu.SemaphoreType.DMA,
        ],
    )
    def k(d_hbm, a_hbm, t2_hbm, dbuf, abuf, t2buf, isem, osem):
        wid = lax.axis_index("s") * 2 + lax.axis_index("c")
        iota = lax.iota(jnp.int32, 16)

        def plane_body(i, carry):
            plane = wid * (RESO // NW) + i
            p0 = plane * PL
            hs = [pltpu.async_copy(d_hbm.at[pl.ds(p0, PL)],
                                   dbuf.at[pl.ds(0, PL)], isem)]
            dlo = jnp.minimum(p0 + PL, RESO ** 3 - 8)
            hs.append(pltpu.async_copy(d_hbm.at[pl.ds(dlo, 8)],
                                       dbuf.at[pl.ds(PL, 8)], isem))
            for c in range(3):
                hs.append(pltpu.async_copy(
                    a_hbm.at[pl.ds((plane * 3 + c) * PL, PL + 8)],
                    abuf.at[c], isem))
            for h in hs:
                h.wait()

            def build_chunk(kc, tb):
                for g in range(C2 // 16):
                    o = kc * C2 + g * 16
                    lvec = iota + g * 16
                    vecs = [dbuf[pl.ds(o, 16)]]
                    vecs += [abuf[c, pl.ds(o, 16)] for c in range(3)]
                    vecs += [dbuf[pl.ds(o + 1, 16)]]
                    vecs += [abuf[c, pl.ds(o + 1, 16)] for c in range(3)]
                    for s in range(8):
                        plsc.store_scatter(
                            tb, [lvec, jnp.full((16,), s, jnp.int32)], vecs[s])
                pltpu.async_copy(tb, t2_hbm.at[pl.ds(p0 + kc * C2, C2)], osem)

            def pair(kc2, carry2):
                for b in range(2):
                    tb = t2buf.at[b]

                    @pl.when(kc2 > 0)
                    def _():
                        pltpu.make_async_copy(
                            tb, t2_hbm.at[pl.ds(p0, C2)], osem).wait()

                    build_chunk(kc2 * 2 + b, tb)
                return carry2

            lax.fori_loop(0, PL // C2 // 2, pair, 0)
            for _ in range(2):
                pltpu.make_async_copy(
                    t2buf.at[0], t2_hbm.at[pl.ds(p0, C2)], osem).wait()
            return carry

        lax.fori_loop(0, RESO // NW, plane_body, 0)

    return k(d_flat, act_flat)


def _sc_gather(table2, base4, slot8, w8):
    mesh = plsc.VectorSubcoreMesh(core_axis_name="c", subcore_axis_name="s")

    @functools.partial(
        pl.kernel,
        mesh=mesh,
        compiler_params=pltpu.CompilerParams(
            needs_layout_passes=False, use_tc_tiling_on_sc=False),
        out_type=jax.ShapeDtypeStruct((4, P), jnp.float32),
        scratch_types=[
            pltpu.VMEM((4, NBLK, 128), jnp.int32),
            pltpu.VMEM((8, NBLK, 128), jnp.int32),
            pltpu.VMEM((8, NBLK, 128), jnp.float32),
            pltpu.VMEM((4, NBLK, 128, 8), jnp.float32),
            pltpu.VMEM((4, CH), jnp.float32),
            pltpu.SemaphoreType.DMA,
        ],
    )
    def k(tab_hbm, base_hbm, slot_hbm, w_hbm, out_hbm,
          base_v, slot_v, w_v, rows_v, out_v, gsem):
        wid = lax.axis_index("s") * 2 + lax.axis_index("c")
        base_blk = wid * (PPW // 128)
        iota = lax.iota(jnp.int32, 16)
        lvecs = [iota + m * 16 for m in range(8)]

        def chunk(ci, carry):
            blk = base_blk + ci * NBLK
            pltpu.sync_copy(base_hbm.at[:, pl.ds(blk, NBLK)], base_v)
            pltpu.sync_copy(slot_hbm.at[:, pl.ds(blk, NBLK)], slot_v)
            pltpu.sync_copy(w_hbm.at[:, pl.ds(blk, NBLK)], w_v)
            handles = []
            for cu in range(4):
                for kb in range(NBLK):
                    handles.append(pltpu.async_copy(
                        tab_hbm.at[base_v.at[cu, kb]], rows_v.at[cu, kb], gsem))
            for h in handles:
                h.wait()
            for g in range(CH // 16):
                kb = g // 8
                lvec = lvecs[g % 8]
                kbv = jnp.full((16,), kb, jnp.int32)
                accs = [jnp.zeros((16,), jnp.float32) for _ in range(4)]
                for u in range(8):
                    uv = jnp.full((16,), u, jnp.int32)
                    cuv = jnp.full((16,), u & 3, jnp.int32)
                    slotv = plsc.load_gather(slot_v, [uv, kbv, lvec])
                    wv = plsc.load_gather(w_v, [uv, kbv, lvec])
                    for c in range(4):
                        val = plsc.load_gather(
                            rows_v, [cuv, kbv, lvec, slotv + c])
                        prod = wv * val
                        accs[c] = accs[c] + prod
                for c in range(4):
                    out_v[c, pl.ds(g * 16, 16)] = accs[c]
            pt0 = wid * PPW + ci * CH
            pltpu.sync_copy(out_v, out_hbm.at[:, pl.ds(pt0, CH)])
            return carry

        lax.fori_loop(0, NCHUNK, chunk, 0)

    return k(table2, base4, slot8, w8)


def _tc_finish(acc4, rays_t, dt):
    JB = 8

    def body(acc_ref, rays_ref, dt_ref, o_ref):
        dens = acc_ref[0]
        rx = rays_ref[3:4, :]
        ry = rays_ref[4:5, :]
        rz = rays_ref[5:6, :]
        norm = jnp.sqrt(rx * rx + ry * ry + rz * rz)
        dt_s = dt_ref[0, 0]
        sigma_a = jax.nn.softplus(dens - 1)
        a = 1 - jnp.exp(-sigma_a * (dt_s * norm))
        b = 1 - jnp.exp(-sigma_a * (1e10 * norm))
        q = 1 - a + 1e-10
        q2 = q * q
        q4 = q2 * q2
        q8 = q4 * q4
        q16 = q8 * q8
        q32 = q16 * q16
        q63 = q32 * q16 * q8 * q4 * q2 * q
        s63 = jnp.where(jnp.abs(1 - q) > 1e-9, (1 - q63) / (1 - q), 63.0)
        o_ref[...] = a * s63 + b * q63

    return pl.pallas_call(
        body,
        grid=(STEPS // JB,),
        in_specs=[
            pl.BlockSpec((4, JB, N_RAYS), lambda j: (0, j, 0)),
            pl.BlockSpec((6, N_RAYS), lambda j: (0, 0)),
            pl.BlockSpec((1, 1), lambda j: (0, 0), memory_space=pltpu.SMEM),
        ],
        out_specs=pl.BlockSpec((JB, N_RAYS), lambda j: (j, 0)),
        out_shape=jax.ShapeDtypeStruct((STEPS, N_RAYS), jnp.float32),
    )(acc4, rays_t, dt.reshape(1, 1))


def _sc_finalize(acc, wtot):
    mesh = plsc.VectorSubcoreMesh(core_axis_name="c", subcore_axis_name="s")

    @functools.partial(
        pl.kernel,
        mesh=mesh,
        compiler_params=pltpu.CompilerParams(
            needs_layout_passes=False, use_tc_tiling_on_sc=False),
        out_type=jax.ShapeDtypeStruct((1, STEPS, N_RAYS, 3), jnp.float32),
        scratch_types=[
            pltpu.VMEM((CH,), jnp.float32),
            pltpu.VMEM((3, CH), jnp.float32),
            pltpu.VMEM((CH, 3), jnp.float32),
        ],
    )
    def k(acc_hbm, w_hbm, out_hbm, w_v, rgb_v, o_v):
        wid = lax.axis_index("s") * 2 + lax.axis_index("c")
        iota = lax.iota(jnp.int32, 16)

        def chunk(ci, carry):
            jj = 2 * wid + ci // 8
            n0 = (ci % 8) * CH
            pt0 = wid * PPW + ci * CH
            pltpu.sync_copy(w_hbm.at[jj, pl.ds(n0, CH)], w_v)
            pltpu.sync_copy(acc_hbm.at[pl.ds(1, 3), pl.ds(pt0, CH)], rgb_v)
            for g in range(CH // 16):
                lvec = iota + g * 16
                wv = w_v[pl.ds(g * 16, 16)]
                for c in range(3):
                    rv = rgb_v[c, pl.ds(g * 16, 16)]
                    plsc.store_scatter(
                        o_v, [lvec, jnp.full((16,), c, jnp.int32)], wv * rv)
            pltpu.sync_copy(o_v, out_hbm.at[0, jj, pl.ds(n0, CH)])
            return carry

        lax.fori_loop(0, NCHUNK, chunk, 0)

    return k(acc, wtot)


def kernel(rays, densities, rgb):
    ts = jnp.linspace(T_NEAR, T_FAR, STEPS, dtype=rays.dtype)
    dt = jnp.clip(ts[1] - ts[0], 1e-5, None)
    rays_t = rays.T
    act_rgb = jax.nn.sigmoid(rgb) * (1 + 2e-3) - 1e-3
    d_flat = densities.reshape(-1)
    act_flat = jnp.transpose(act_rgb, (0, 3, 1, 2)).reshape(-1)
    table2 = _sc_build(d_flat, act_flat)

    base4, slot8, w8 = _tc_prep(rays_t, ts[:, None])
    acc = _sc_gather(table2,
                     base4.reshape(4, P // 128, 128),
                     slot8.reshape(8, P // 128, 128),
                     w8.reshape(8, P // 128, 128))
    wtot = _tc_finish(acc.reshape(4, STEPS, N_RAYS), rays_t, dt)
    return _sc_finalize(acc, wtot)


# speedup vs baseline: 2.5962x; 1.0292x over previous
import functools

import jax
import jax.numpy as jnp
from jax import lax
from jax.experimental import pallas as pl
from jax.experimental.pallas import tpu as pltpu
from jax.experimental.pallas import tpu_sc as plsc

RESO = 128
OUT = 3
G_RAD = 1.3
T_NEAR = 0.2
T_FAR = 2.0
STEPS = 64
VOXEL_LEN = G_RAD * 2 / RESO
N_RAYS = 4096
EPS = 1e-10

P = STEPS * N_RAYS
NW = 32
PPW = P // NW
CH = 512
NCHUNK = PPW // CH
NBLK = CH // 128


def _tc_prep(rays_t, ts2):
    JB = 8

    def body(rays_ref, ts_ref, base_ref, slot_ref, w_ref):
        t = ts_ref[...]
        pts = []
        for d in range(3):
            ro = rays_ref[d:d + 1, :]
            rd = rays_ref[d + 3:d + 4, :]
            pts.append(ro + t * rd)
        ilo, ihi, tx = [], [], []
        for d in range(3):
            p = pts[d]
            nlo = jnp.clip(-0.5 * VOXEL_LEN + p, -G_RAD, G_RAD)
            nhi = jnp.clip(0.5 * VOXEL_LEN + p, -G_RAD, G_RAD)
            clo = jnp.clip((jnp.floor(nlo / VOXEL_LEN + EPS) + 0.5) * VOXEL_LEN,
                           -(G_RAD - VOXEL_LEN / 2), G_RAD - VOXEL_LEN / 2)
            chi = jnp.clip((jnp.floor(nhi / VOXEL_LEN + EPS) + 0.5) * VOXEL_LEN,
                           -(G_RAD - VOXEL_LEN / 2), G_RAD - VOXEL_LEN / 2)
            ilo.append(jnp.floor(clo / VOXEL_LEN + EPS).astype(jnp.int32) + RESO // 2)
            ihi.append(jnp.floor(chi / VOXEL_LEN + EPS).astype(jnp.int32) + RESO // 2)
            x = (p - clo) / VOXEL_LEN
            tx.append((1 - x, x))
        zbase = jnp.minimum(ilo[2], RESO - 2)
        for cu in range(4):
            bx, by = cu & 1, (cu >> 1) & 1
            ix = ihi[0] if bx else ilo[0]
            iy = ihi[1] if by else ilo[1]
            base_ref[cu] = (ix * RESO + iy) * RESO + zbase
        for u in range(8):
            bx, by, bz = u & 1, (u >> 1) & 1, (u >> 2) & 1
            iz = ihi[2] if bz else ilo[2]
            slot_ref[u] = jnp.where(iz == zbase, 0, 4).astype(jnp.int32)
            w_ref[u] = tx[0][bx] * tx[1][by] * tx[2][bz]

    return pl.pallas_call(
        body,
        grid=(STEPS // JB,),
        in_specs=[
            pl.BlockSpec((6, N_RAYS), lambda j: (0, 0)),
            pl.BlockSpec((JB, 1), lambda j: (j, 0)),
        ],
        out_specs=[
            pl.BlockSpec((4, JB, N_RAYS), lambda j: (0, j, 0)),
            pl.BlockSpec((8, JB, N_RAYS), lambda j: (0, j, 0)),
            pl.BlockSpec((8, JB, N_RAYS), lambda j: (0, j, 0)),
        ],
        out_shape=[
            jax.ShapeDtypeStruct((4, STEPS, N_RAYS), jnp.int32),
            jax.ShapeDtypeStruct((8, STEPS, N_RAYS), jnp.int32),
            jax.ShapeDtypeStruct((8, STEPS, N_RAYS), jnp.float32),
        ],
    )(rays_t, ts2)


def _sc_build(d_flat, act_flat):
    PL = RESO * RESO
    C2 = 2048
    NC = (RESO // NW) * (PL // C2)
    mesh = plsc.VectorSubcoreMesh(core_axis_name="c", subcore_axis_name="s")

    del NC

    @functools.partial(
        pl.kernel,
        mesh=mesh,
        compiler_params=pltpu.CompilerParams(
            needs_layout_passes=False, use_tc_tiling_on_sc=False),
        out_type=jax.ShapeDtypeStruct((RESO ** 3, 8), jnp.float32),
        scratch_types=[
            pltpu.VMEM((PL + 8,), jnp.float32),
            pltpu.VMEM((3, PL + 8), jnp.float32),
            pltpu.VMEM((2, C2, 8), jnp.float32),
            pltpu.SemaphoreType.DMA,
            pltpu.SemaphoreType.DMA,
        ],
    )
    def k(d_hbm, a_hbm, t2_hbm, dbuf, abuf, t2buf, isem, osem):
        wid = lax.axis_index("s") * 2 + lax.axis_index("c")
        iota = lax.iota(jnp.int32, 16)

        def plane_body(i, carry):
            plane = wid * (RESO // NW) + i
            p0 = plane * PL
            hs = [pltpu.async_copy(d_hbm.at[pl.ds(p0, PL)],
                                   dbuf.at[pl.ds(0, PL)], isem)]
            dlo = jnp.minimum(p0 + PL, RESO ** 3 - 8)
            hs.append(pltpu.async_copy(d_hbm.at[pl.ds(dlo, 8)],
                                       dbuf.at[pl.ds(PL, 8)], isem))
            for c in range(3):
                hs.append(pltpu.async_copy(
                    a_hbm.at[pl.ds((plane * 3 + c) * PL, PL + 8)],
                    abuf.at[c], isem))
            for h in hs:
                h.wait()

            def build_chunk(kc, tb):
                for g in range(C2 // 16):
                    o = kc * C2 + g * 16
                    lvec = iota + g * 16
                    vecs = [dbuf[pl.ds(o, 16)]]
                    vecs += [abuf[c, pl.ds(o, 16)] for c in range(3)]
                    vecs += [dbuf[pl.ds(o + 1, 16)]]
                    vecs += [abuf[c, pl.ds(o + 1, 16)] for c in range(3)]
                    for s in range(8):
                        plsc.store_scatter(
                            tb, [lvec, jnp.full((16,), s, jnp.int32)], vecs[s])
                pltpu.async_copy(tb, t2_hbm.at[pl.ds(p0 + kc * C2, C2)], osem)

            def pair(kc2, carry2):
                for b in range(2):
                    tb = t2buf.at[b]

                    @pl.when(kc2 > 0)
                    def _():
                        pltpu.make_async_copy(
                            tb, t2_hbm.at[pl.ds(p0, C2)], osem).wait()

                    build_chunk(kc2 * 2 + b, tb)
                return carry2

            lax.fori_loop(0, PL // C2 // 2, pair, 0)
            for _ in range(2):
                pltpu.make_async_copy(
                    t2buf.at[0], t2_hbm.at[pl.ds(p0, C2)], osem).wait()
            return carry

        lax.fori_loop(0, RESO // NW, plane_body, 0)

    return k(d_flat, act_flat)


def _sc_gather(table2, base4, slot8, w8):
    mesh = plsc.VectorSubcoreMesh(core_axis_name="c", subcore_axis_name="s")

    @functools.partial(
        pl.kernel,
        mesh=mesh,
        compiler_params=pltpu.CompilerParams(
            needs_layout_passes=False, use_tc_tiling_on_sc=False),
        out_type=jax.ShapeDtypeStruct((4, P), jnp.float32),
        scratch_types=[
            pltpu.VMEM((4, NBLK, 128), jnp.int32),
            pltpu.VMEM((8, NBLK, 128), jnp.int32),
            pltpu.VMEM((8, NBLK, 128), jnp.float32),
            pltpu.VMEM((4, NBLK, 128, 8), jnp.float32),
            pltpu.VMEM((4, CH), jnp.float32),
            pltpu.SemaphoreType.DMA,
        ],
    )
    def k(tab_hbm, base_hbm, slot_hbm, w_hbm, out_hbm,
          base_v, slot_v, w_v, rows_v, out_v, gsem):
        wid = lax.axis_index("s") * 2 + lax.axis_index("c")
        base_blk = wid * (PPW // 128)
        iota = lax.iota(jnp.int32, 16)
        lvecs = [iota + m * 16 for m in range(8)]

        def chunk(ci, carry):
            blk = base_blk + ci * NBLK
            hs = [pltpu.async_copy(base_hbm.at[:, pl.ds(blk, NBLK)], base_v, gsem),
                  pltpu.async_copy(slot_hbm.at[:, pl.ds(blk, NBLK)], slot_v, gsem),
                  pltpu.async_copy(w_hbm.at[:, pl.ds(blk, NBLK)], w_v, gsem)]
            hs[0].wait()
            handles = []
            for cu in range(4):
                for kb in range(NBLK):
                    handles.append(pltpu.async_copy(
                        tab_hbm.at[base_v.at[cu, kb]], rows_v.at[cu, kb], gsem))
            hs[1].wait()
            hs[2].wait()
            for h in handles:
                h.wait()
            for g in range(CH // 16):
                kb = g // 8
                lvec = lvecs[g % 8]
                kbv = jnp.full((16,), kb, jnp.int32)
                accs = [jnp.zeros((16,), jnp.float32) for _ in range(4)]
                for u in range(8):
                    uv = jnp.full((16,), u, jnp.int32)
                    cuv = jnp.full((16,), u & 3, jnp.int32)
                    slotv = plsc.load_gather(slot_v, [uv, kbv, lvec])
                    wv = plsc.load_gather(w_v, [uv, kbv, lvec])
                    for c in range(4):
                        val = plsc.load_gather(
                            rows_v, [cuv, kbv, lvec, slotv + c])
                        prod = wv * val
                        accs[c] = accs[c] + prod
                for c in range(4):
                    out_v[c, pl.ds(g * 16, 16)] = accs[c]
            pt0 = wid * PPW + ci * CH
            pltpu.sync_copy(out_v, out_hbm.at[:, pl.ds(pt0, CH)])
            return carry

        lax.fori_loop(0, NCHUNK, chunk, 0)

    return k(table2, base4, slot8, w8)


def _tc_finish(acc4, rays_t, dt):
    JB = 8

    def body(acc_ref, rays_ref, dt_ref, o_ref):
        dens = acc_ref[0]
        rx = rays_ref[3:4, :]
        ry = rays_ref[4:5, :]
        rz = rays_ref[5:6, :]
        norm = jnp.sqrt(rx * rx + ry * ry + rz * rz)
        dt_s = dt_ref[0, 0]
        sigma_a = jax.nn.softplus(dens - 1)
        a = 1 - jnp.exp(-sigma_a * (dt_s * norm))
        b = 1 - jnp.exp(-sigma_a * (1e10 * norm))
        q = 1 - a + 1e-10
        q2 = q * q
        q4 = q2 * q2
        q8 = q4 * q4
        q16 = q8 * q8
        q32 = q16 * q16
        q63 = q32 * q16 * q8 * q4 * q2 * q
        s63 = jnp.where(jnp.abs(1 - q) > 1e-9, (1 - q63) / (1 - q), 63.0)
        o_ref[...] = a * s63 + b * q63

    return pl.pallas_call(
        body,
        grid=(STEPS // JB,),
        in_specs=[
            pl.BlockSpec((4, JB, N_RAYS), lambda j: (0, j, 0)),
            pl.BlockSpec((6, N_RAYS), lambda j: (0, 0)),
            pl.BlockSpec((1, 1), lambda j: (0, 0), memory_space=pltpu.SMEM),
        ],
        out_specs=pl.BlockSpec((JB, N_RAYS), lambda j: (j, 0)),
        out_shape=jax.ShapeDtypeStruct((STEPS, N_RAYS), jnp.float32),
    )(acc4, rays_t, dt.reshape(1, 1))


def _sc_finalize(acc, wtot):
    mesh = plsc.VectorSubcoreMesh(core_axis_name="c", subcore_axis_name="s")

    @functools.partial(
        pl.kernel,
        mesh=mesh,
        compiler_params=pltpu.CompilerParams(
            needs_layout_passes=False, use_tc_tiling_on_sc=False),
        out_type=jax.ShapeDtypeStruct((1, STEPS, N_RAYS, 3), jnp.float32),
        scratch_types=[
            pltpu.VMEM((CH,), jnp.float32),
            pltpu.VMEM((3, CH), jnp.float32),
            pltpu.VMEM((CH, 3), jnp.float32),
        ],
    )
    def k(acc_hbm, w_hbm, out_hbm, w_v, rgb_v, o_v):
        wid = lax.axis_index("s") * 2 + lax.axis_index("c")
        iota = lax.iota(jnp.int32, 16)

        def chunk(ci, carry):
            jj = 2 * wid + ci // 8
            n0 = (ci % 8) * CH
            pt0 = wid * PPW + ci * CH
            pltpu.sync_copy(w_hbm.at[jj, pl.ds(n0, CH)], w_v)
            pltpu.sync_copy(acc_hbm.at[pl.ds(1, 3), pl.ds(pt0, CH)], rgb_v)
            for g in range(CH // 16):
                lvec = iota + g * 16
                wv = w_v[pl.ds(g * 16, 16)]
                for c in range(3):
                    rv = rgb_v[c, pl.ds(g * 16, 16)]
                    plsc.store_scatter(
                        o_v, [lvec, jnp.full((16,), c, jnp.int32)], wv * rv)
            pltpu.sync_copy(o_v, out_hbm.at[0, jj, pl.ds(n0, CH)])
            return carry

        lax.fori_loop(0, NCHUNK, chunk, 0)

    return k(acc, wtot)


def kernel(rays, densities, rgb):
    ts = jnp.linspace(T_NEAR, T_FAR, STEPS, dtype=rays.dtype)
    dt = jnp.clip(ts[1] - ts[0], 1e-5, None)
    rays_t = rays.T
    act_rgb = jax.nn.sigmoid(rgb) * (1 + 2e-3) - 1e-3
    d_flat = densities.reshape(-1)
    act_flat = jnp.transpose(act_rgb, (0, 3, 1, 2)).reshape(-1)
    table2 = _sc_build(d_flat, act_flat)

    base4, slot8, w8 = _tc_prep(rays_t, ts[:, None])
    acc = _sc_gather(table2,
                     base4.reshape(4, P // 128, 128),
                     slot8.reshape(8, P // 128, 128),
                     w8.reshape(8, P // 128, 128))
    wtot = _tc_finish(acc.reshape(4, STEPS, N_RAYS), rays_t, dt)
    return _sc_finalize(acc, wtot)
